# Initial kernel scaffold; baseline (speedup 1.0000x reference)
import jax
import jax.numpy as jnp
from jax.experimental import pallas as pl


def kernel(x, edge_index, y, batch, epoch, params):
    raise NotImplementedError("write your pallas kernel here")


# speedup vs baseline: 17.9646x; 17.9646x over previous
import functools

import jax
import jax.numpy as jnp
from jax import lax
from jax.experimental import pallas as pl
from jax.experimental.pallas import tpu as pltpu
from jax.experimental.pallas import tpu_sc as plsc

NN = 50000
NE = 800000
NG = 500
NEG = 0.2
EPS = 1e-16

RB = 1000
GRID = NN // RB

CK = 128
NCHUNK = NE // CK
NSUB = 16
RPS = 3128
RPS_LAST = NN - 15 * RPS

F32 = jnp.float32
I32 = jnp.int32


def _halves(fout):
  d0 = (fout + 1) // 2
  return d0, fout - d0


def _pack_haug(haug, fout, ht_ref, es2_ref, ed_ref, gmax_ref):
  h = haug[:, :fout]
  es = haug[:, fout]
  ed = haug[:, fout + 1]
  d0, d1 = _halves(fout)

  def _padded(part, d):
    if d == 32:
      return part
    return jnp.concatenate([part, jnp.zeros((part.shape[0], 32 - d), F32)],
                           axis=1)

  ht_ref[0] = _padded(h[:, :d0], d0)
  ht_ref[1] = _padded(h[:, d0:], d1)
  es2_ref[0, 0, 0] = es
  es2_ref[1, 0, 0] = es
  ed_ref[0, 0] = ed
  i = pl.program_id(0)

  @pl.when(i == 0)
  def _():
    gmax_ref[...] = jnp.full((16,), -jnp.inf, F32)

  gmax_ref[...] = jnp.maximum(gmax_ref[...], jnp.max(es))


def _tc_first_body(x_ref, w_ref, ht_ref, es2_ref, ed_ref, gmax_ref, *, fout):
  haug = jnp.dot(x_ref[...], w_ref[...], preferred_element_type=F32)
  _pack_haug(haug, fout, ht_ref, es2_ref, ed_ref, gmax_ref)


def _finalize_x(acc_ref, den_ref, b_ref, fin, relu):
  d0, d1 = _halves(fin)
  den = den_ref[0, 0] + EPS
  x = jnp.concatenate([acc_ref[0][:, :d0], acc_ref[1][:, :d1]], axis=1)
  x = x / den[:, None] + b_ref[...]
  if relu:
    x = jnp.maximum(x, 0.0)
  return x


def _tc_mid_body(acc_ref, den_ref, b_ref, w_ref,
                 ht_ref, es2_ref, ed_ref, gmax_ref, *rest, fin, fout,
                 want_x):
  x = _finalize_x(acc_ref, den_ref, b_ref, fin, True)
  if want_x:
    rest[0][...] = x
  haug = jnp.dot(x, w_ref[...], preferred_element_type=F32)
  _pack_haug(haug, fout, ht_ref, es2_ref, ed_ref, gmax_ref)


def _tc_mask3_body(acc_ref, den_ref, b_ref, lat_ref, w_ref,
                   ht_ref, es2_ref, ed_ref, gmax_ref, msk_ref, out2_ref,
                   *, fout):
  den = den_ref[0, 0] + EPS
  z0 = acc_ref[0][:, 0] / den + b_ref[0]
  z1 = acc_ref[1][:, 0] / den + b_ref[1]
  t = z1 - z0
  soft = 1.0 / (1.0 + jnp.exp(-t))
  hard = (t > 0.0).astype(F32)
  lat = lat_ref[...]
  msk = lat * soft[:, None]
  msk_ref[...] = msk
  out2_ref[...] = lat * hard[:, None]
  haug = jnp.dot(msk, w_ref[...], preferred_element_type=F32)
  _pack_haug(haug, fout, ht_ref, es2_ref, ed_ref, gmax_ref)


def _tc_last_body(acc_ref, den_ref, b_ref, d3_ref, *, fin):
  d3_ref[...] = _finalize_x(acc_ref, den_ref, b_ref, fin, False)


def _tc_pool_body(msk_ref, batch_ref, out_ref):
  i = pl.program_id(0)
  seg = batch_ref[0, 0]
  gidx = lax.broadcasted_iota(I32, (RB, NG), 1)
  oh = (seg[:, None] == gidx).astype(F32)
  x1 = jnp.concatenate([msk_ref[...], jnp.ones((RB, 1), F32)], axis=1)
  contrib = lax.dot_general(oh, x1, (((0,), (0,)), ((), ())),
                            preferred_element_type=F32)

  @pl.when(i == 0)
  def _():
    out_ref[...] = jnp.zeros_like(out_ref)

  out_ref[...] += contrib


def _tc_head_body(pool_ref, w1_ref, b1_ref, w2_ref, b2_ref, c_ref):
  pool = pool_ref[...]
  counts = jnp.maximum(pool[:, 64], 1.0)
  c = pool[:, :64] / counts[:, None]
  c = jnp.dot(c, w1_ref[...], preferred_element_type=F32) + b1_ref[...]
  c = jnp.dot(c, w2_ref[...], preferred_element_type=F32) + b2_ref[...]
  c_ref[...] = c


def _full(shape):
  return pl.BlockSpec(shape, lambda i: tuple(0 for _ in shape))


_HT_SPEC = pl.BlockSpec((2, RB, 32), lambda i: (0, i, 0))
_ES2_SPEC = pl.BlockSpec((2, 1, 1, RB), lambda i: (0, i, 0, 0))
_ED_SPEC = pl.BlockSpec((1, 1, RB), lambda i: (i, 0, 0))
_GMAX_SPEC = pl.BlockSpec((16,), lambda i: (0,))
_ACC_SPEC = pl.BlockSpec((2, RB, 32), lambda i: (0, i, 0))
_DEN_SPEC = pl.BlockSpec((1, 1, RB), lambda i: (i, 0, 0))
_X_SPEC = lambda d: pl.BlockSpec((RB, d), lambda i: (i, 0))

_HAUG_OUT = lambda: [
    jax.ShapeDtypeStruct((2, NN, 32), F32),
    jax.ShapeDtypeStruct((2, GRID, 1, RB), F32),
    jax.ShapeDtypeStruct((GRID, 1, RB), F32),
    jax.ShapeDtypeStruct((16,), F32),
]
_HAUG_SPECS = lambda: [_HT_SPEC, _ES2_SPEC, _ED_SPEC, _GMAX_SPEC]


def _tc_first(x, w_aug, fout):
  f = functools.partial(_tc_first_body, fout=fout)
  return pl.pallas_call(
      f, grid=(GRID,),
      in_specs=[_X_SPEC(x.shape[1]), _full(w_aug.shape)],
      out_specs=_HAUG_SPECS(),
      out_shape=_HAUG_OUT(),
  )(x, w_aug)


def _tc_mid(acc, den, b, w_aug, fin, fout, want_x=False):
  f = functools.partial(_tc_mid_body, fin=fin, fout=fout, want_x=want_x)
  out_specs = _HAUG_SPECS()
  out_shape = _HAUG_OUT()
  if want_x:
    out_specs.append(_X_SPEC(fin))
    out_shape.append(jax.ShapeDtypeStruct((NN, fin), F32))
  return pl.pallas_call(
      f, grid=(GRID,),
      in_specs=[_ACC_SPEC, _DEN_SPEC, _full(b.shape), _full(w_aug.shape)],
      out_specs=out_specs,
      out_shape=out_shape,
  )(acc, den, b, w_aug)


def _tc_mask3(acc, den, b, latent, w_aug, fout):
  f = functools.partial(_tc_mask3_body, fout=fout)
  out_specs = _HAUG_SPECS() + [_X_SPEC(64), _X_SPEC(64)]
  out_shape = _HAUG_OUT() + [jax.ShapeDtypeStruct((NN, 64), F32),
                             jax.ShapeDtypeStruct((NN, 64), F32)]
  return pl.pallas_call(
      f, grid=(GRID,),
      in_specs=[_ACC_SPEC, _DEN_SPEC, _full(b.shape), _X_SPEC(64),
                _full(w_aug.shape)],
      out_specs=out_specs,
      out_shape=out_shape,
  )(acc, den, b, latent, w_aug)


def _tc_last(acc, den, b, fin):
  f = functools.partial(_tc_last_body, fin=fin)
  return pl.pallas_call(
      f, grid=(GRID,),
      in_specs=[_ACC_SPEC, _DEN_SPEC, _full(b.shape)],
      out_specs=_X_SPEC(fin),
      out_shape=jax.ShapeDtypeStruct((NN, fin), F32),
  )(acc, den, b)


def _tc_pool(msk, batch):
  return pl.pallas_call(
      _tc_pool_body, grid=(GRID,),
      in_specs=[_X_SPEC(64), pl.BlockSpec((1, 1, RB), lambda i: (i, 0, 0))],
      out_specs=pl.BlockSpec((NG, 65), lambda i: (0, 0)),
      out_shape=jax.ShapeDtypeStruct((NG, 65), F32),
  )(msk, batch)


def _tc_head(pool, w1, b1, w2, b2):
  return pl.pallas_call(
      _tc_head_body, grid=(1,),
      in_specs=[_full(pool.shape), _full(w1.shape), _full(b1.shape),
                _full(w2.shape), _full(b2.shape)],
      out_specs=_full((NG, 11)),
      out_shape=jax.ShapeDtypeStruct((NG, 11), F32),
  )(pool, w1, b1, w2, b2)


def _sc_edge_body(ht, es2, ed, gmax, ei, z32, z1,
                  acc_out, den_out,
                  srcv, dstv, esg, edg, rowg, pv, gmx,
                  acc_sh, den_sh, gsem, ssem):
  c = lax.axis_index("c")
  s = lax.axis_index("s")
  row0 = s * RPS

  @pl.when(s < 15)
  def _():
    pltpu.sync_copy(z32, acc_sh.at[pl.ds(row0, RPS), :])
    pltpu.sync_copy(z1, den_sh.at[pl.ds(row0, RPS), :])

  @pl.when(s == 15)
  def _():
    pltpu.sync_copy(z32.at[pl.ds(0, RPS_LAST), :],
                    acc_sh.at[pl.ds(row0, RPS_LAST), :])
    pltpu.sync_copy(z1.at[pl.ds(0, RPS_LAST), :],
                    den_sh.at[pl.ds(row0, RPS_LAST), :])

  pltpu.sync_copy(gmax, gmx)
  plsc.subcore_barrier()

  gm = gmx[...]
  coff = c * NN
  ntrip = jnp.where(s < (NCHUNK % NSUB), NCHUNK // NSUB + 1, NCHUNK // NSUB)

  def chunk_body(k, carry):
    off = (s + NSUB * k) * CK
    pltpu.sync_copy(ei.at[0, pl.ds(off, CK)], srcv)
    pltpu.sync_copy(ei.at[1, pl.ds(off, CK)], dstv)
    for g in range(CK // 16):
      sl = pl.ds(g * 16, 16)
      srcv[sl] = srcv[sl] + coff
    d1 = pltpu.async_copy(es2.at[srcv], esg, gsem)
    d2 = pltpu.async_copy(ed.at[dstv], edg, gsem)
    d3 = pltpu.async_copy(ht.at[srcv], rowg, gsem)
    d1.wait()
    d2.wait()
    d3.wait()
    zcol = jnp.zeros((16,), I32)
    base = lax.iota(I32, 16)
    for g in range(CK // 16):
      sl = pl.ds(g * 16, 16)
      a = esg[sl]
      bb = edg[sl]
      t = a + bb
      e = jnp.maximum(t, NEG * t)
      u = gm + bb
      cb = jnp.maximum(u, NEG * u)
      p = jnp.exp(e - cb)
      plsc.store_scatter(pv, [base + (g * 16), zcol], p)

    def row_body(r, carry2):
      bc = plsc.load_gather(pv, [jnp.full((16,), r, I32),
                                 jnp.zeros((16,), I32)])
      rowg[r, pl.ds(0, 16)] = rowg[r, pl.ds(0, 16)] * bc
      rowg[r, pl.ds(16, 16)] = rowg[r, pl.ds(16, 16)] * bc
      return carry2

    lax.fori_loop(0, CK, row_body, 0)
    e1 = pltpu.async_copy(pv, den_sh.at[dstv], ssem, add=True)
    e2 = pltpu.async_copy(rowg, acc_sh.at[dstv], ssem, add=True)
    e1.wait()
    e2.wait()
    return carry

  lax.fori_loop(0, ntrip, chunk_body, 0)
  plsc.subcore_barrier()
  out0 = coff + row0

  @pl.when(s < 15)
  def _():
    pltpu.sync_copy(acc_sh.at[pl.ds(row0, RPS), :],
                    acc_out.at[pl.ds(out0, RPS), :])
    pltpu.sync_copy(den_sh.at[pl.ds(row0, RPS), :],
                    den_out.at[pl.ds(out0, RPS), :])

  @pl.when(s == 15)
  def _():
    pltpu.sync_copy(acc_sh.at[pl.ds(row0, RPS_LAST), :],
                    acc_out.at[pl.ds(out0, RPS_LAST), :])
    pltpu.sync_copy(den_sh.at[pl.ds(row0, RPS_LAST), :],
                    den_out.at[pl.ds(out0, RPS_LAST), :])


@functools.partial(jax.jit, static_argnums=())
def _sc_edge_call(ht, es2, ed, gmax, ei, z32, z8):
  mesh = plsc.VectorSubcoreMesh(core_axis_name="c", subcore_axis_name="s")
  kern = pl.kernel(
      _sc_edge_body,
      out_type=[jax.ShapeDtypeStruct((2 * NN, 32), F32),
                jax.ShapeDtypeStruct((2 * NN, 1), F32)],
      mesh=mesh,
      compiler_params=pltpu.CompilerParams(needs_layout_passes=False,
                                           use_tc_tiling_on_sc=False),
      scratch_types=[
          pltpu.VMEM((CK,), I32),
          pltpu.VMEM((CK,), I32),
          pltpu.VMEM((CK,), F32),
          pltpu.VMEM((CK,), F32),
          pltpu.VMEM((CK, 32), F32),
          pltpu.VMEM((CK, 1), F32),
          pltpu.VMEM((16,), F32),
          pltpu.VMEM_SHARED((NN, 32), F32),
          pltpu.VMEM_SHARED((NN, 1), F32),
          pltpu.SemaphoreType.DMA,
          pltpu.SemaphoreType.DMA,
      ],
  )
  return kern(ht, es2, ed, gmax, ei, z32, z8)


def _sc_edge(ht, es2, ed, gmax, ei, z32, z1):
  acc, den = _sc_edge_call(ht.reshape(2 * NN, 32), es2.reshape(2 * NN),
                           ed.reshape(NN), gmax, ei, z32, z1)
  return acc.reshape(2, NN, 32), den.reshape(2, NN)[0].reshape(GRID, 1, RB)


def _aug(p):
  w, a_src, a_dst, b = p
  return jnp.concatenate(
      [w, (w @ a_src)[:, None], (w @ a_dst)[:, None]], axis=1), b


def kernel(x, edge_index, y, batch, epoch, params):
  del y, epoch
  ei = edge_index.astype(I32)
  z32 = jnp.zeros((RPS, 32), F32)
  z1 = jnp.zeros((RPS, 1), F32)

  wa1, b1 = _aug(params['conv1'])
  wa2, b2 = _aug(params['conv2'])
  wa3, b3 = _aug(params['conv3'])
  wm1, bm1 = _aug(params['mask1'])
  wm2, bm2 = _aug(params['mask2'])
  wm3, bm3 = _aug(params['mask3'])
  wa4, b4 = _aug(params['conv4'])
  wa5, b5 = _aug(params['conv5'])
  wa6, b6 = _aug(params['conv6'])

  def edge(haug):
    ht, es2, ed, gmax = haug[0], haug[1], haug[2], haug[3]
    return _sc_edge(ht, es2, ed, gmax, ei, z32, z1)

  o = _tc_first(x, wa1, 64)
  acc, den = edge(o)
  o = _tc_mid(acc, den, b1, wa2, 64, 64)
  acc, den = edge(o)
  o = _tc_mid(acc, den, b2, wa3, 64, 64)
  acc, den = edge(o)
  o = _tc_mid(acc, den, b3, wm1, 64, 32, want_x=True)
  latent = o[4]
  acc, den = edge(o)
  o = _tc_mid(acc, den, bm1, wm2, 32, 32)
  acc, den = edge(o)
  o = _tc_mid(acc, den, bm2, wm3, 32, 2)
  acc, den = edge(o)
  o = _tc_mask3(acc, den, bm3, latent, wa4, 64)
  msk, out2 = o[4], o[5]
  acc, den = edge(o)
  o = _tc_mid(acc, den, b4, wa5, 64, 64)
  acc, den = edge(o)
  o = _tc_mid(acc, den, b5, wa6, 64, 5)
  acc, den = edge(o)
  d3 = _tc_last(acc, den, b6, 5)

  pool = _tc_pool(msk, batch.astype(I32).reshape(GRID, 1, RB))
  w1, bb1 = params['lin1']
  w2, bb2 = params['lin2']
  c = _tc_head(pool, w1, bb1, w2, bb2)
  return (d3, out2, c)


# speedup vs baseline: 17.9711x; 1.0004x over previous
import functools

import jax
import jax.numpy as jnp
from jax import lax
from jax.experimental import pallas as pl
from jax.experimental.pallas import tpu as pltpu
from jax.experimental.pallas import tpu_sc as plsc

NN = 50000
NE = 800000
NG = 500
NEG = 0.2
EPS = 1e-16

RB = 1000
GRID = NN // RB

CK = 128
NSC = 1
SK = CK * NSC
NCHUNK = NE // SK
NSUB = 16
RPS = 3128
RPS_LAST = NN - 15 * RPS

F32 = jnp.float32
I32 = jnp.int32


def _halves(fout):
  d0 = (fout + 1) // 2
  return d0, fout - d0


def _pack_haug(haug, fout, ht_ref, es2_ref, ed_ref, gmax_ref):
  h = haug[:, :fout]
  es = haug[:, fout]
  ed = haug[:, fout + 1]
  d0, d1 = _halves(fout)

  def _padded(part, d):
    if d == 32:
      return part
    return jnp.concatenate([part, jnp.zeros((part.shape[0], 32 - d), F32)],
                           axis=1)

  ht_ref[0] = _padded(h[:, :d0], d0)
  ht_ref[1] = _padded(h[:, d0:], d1)
  es2_ref[0, 0, 0] = es
  es2_ref[1, 0, 0] = es
  ed_ref[0, 0] = ed
  i = pl.program_id(0)

  @pl.when(i == 0)
  def _():
    gmax_ref[...] = jnp.full((16,), -jnp.inf, F32)

  gmax_ref[...] = jnp.maximum(gmax_ref[...], jnp.max(es))


def _tc_first_body(x_ref, w_ref, ht_ref, es2_ref, ed_ref, gmax_ref, *, fout):
  haug = jnp.dot(x_ref[...], w_ref[...], preferred_element_type=F32)
  _pack_haug(haug, fout, ht_ref, es2_ref, ed_ref, gmax_ref)


def _finalize_x(acc_ref, den_ref, b_ref, fin, relu):
  d0, d1 = _halves(fin)
  den = den_ref[0, 0] + EPS
  x = jnp.concatenate([acc_ref[0][:, :d0], acc_ref[1][:, :d1]], axis=1)
  x = x / den[:, None] + b_ref[...]
  if relu:
    x = jnp.maximum(x, 0.0)
  return x


def _tc_mid_body(acc_ref, den_ref, b_ref, w_ref,
                 ht_ref, es2_ref, ed_ref, gmax_ref, *rest, fin, fout,
                 want_x):
  x = _finalize_x(acc_ref, den_ref, b_ref, fin, True)
  if want_x:
    rest[0][...] = x
  haug = jnp.dot(x, w_ref[...], preferred_element_type=F32)
  _pack_haug(haug, fout, ht_ref, es2_ref, ed_ref, gmax_ref)


def _tc_mask3_body(acc_ref, den_ref, b_ref, lat_ref, w_ref,
                   ht_ref, es2_ref, ed_ref, gmax_ref, msk_ref, out2_ref,
                   *, fout):
  den = den_ref[0, 0] + EPS
  z0 = acc_ref[0][:, 0] / den + b_ref[0]
  z1 = acc_ref[1][:, 0] / den + b_ref[1]
  t = z1 - z0
  soft = 1.0 / (1.0 + jnp.exp(-t))
  hard = (t > 0.0).astype(F32)
  lat = lat_ref[...]
  msk = lat * soft[:, None]
  msk_ref[...] = msk
  out2_ref[...] = lat * hard[:, None]
  haug = jnp.dot(msk, w_ref[...], preferred_element_type=F32)
  _pack_haug(haug, fout, ht_ref, es2_ref, ed_ref, gmax_ref)


def _tc_last_body(acc_ref, den_ref, b_ref, d3_ref, *, fin):
  d3_ref[...] = _finalize_x(acc_ref, den_ref, b_ref, fin, False)


def _tc_pool_body(msk_ref, batch_ref, out_ref):
  i = pl.program_id(0)
  seg = batch_ref[0, 0]
  gidx = lax.broadcasted_iota(I32, (RB, NG), 1)
  oh = (seg[:, None] == gidx).astype(F32)
  x1 = jnp.concatenate([msk_ref[...], jnp.ones((RB, 1), F32)], axis=1)
  contrib = lax.dot_general(oh, x1, (((0,), (0,)), ((), ())),
                            preferred_element_type=F32)

  @pl.when(i == 0)
  def _():
    out_ref[...] = jnp.zeros_like(out_ref)

  out_ref[...] += contrib


def _tc_head_body(pool_ref, w1_ref, b1_ref, w2_ref, b2_ref, c_ref):
  pool = pool_ref[...]
  counts = jnp.maximum(pool[:, 64], 1.0)
  c = pool[:, :64] / counts[:, None]
  c = jnp.dot(c, w1_ref[...], preferred_element_type=F32) + b1_ref[...]
  c = jnp.dot(c, w2_ref[...], preferred_element_type=F32) + b2_ref[...]
  c_ref[...] = c


def _full(shape):
  return pl.BlockSpec(shape, lambda i: tuple(0 for _ in shape))


_HT_SPEC = pl.BlockSpec((2, RB, 32), lambda i: (0, i, 0))
_ES2_SPEC = pl.BlockSpec((2, 1, 1, RB), lambda i: (0, i, 0, 0))
_ED_SPEC = pl.BlockSpec((1, 1, RB), lambda i: (i, 0, 0))
_GMAX_SPEC = pl.BlockSpec((16,), lambda i: (0,))
_ACC_SPEC = pl.BlockSpec((2, RB, 32), lambda i: (0, i, 0))
_DEN_SPEC = pl.BlockSpec((1, 1, RB), lambda i: (i, 0, 0))
_X_SPEC = lambda d: pl.BlockSpec((RB, d), lambda i: (i, 0))

_HAUG_OUT = lambda: [
    jax.ShapeDtypeStruct((2, NN, 32), F32),
    jax.ShapeDtypeStruct((2, GRID, 1, RB), F32),
    jax.ShapeDtypeStruct((GRID, 1, RB), F32),
    jax.ShapeDtypeStruct((16,), F32),
]
_HAUG_SPECS = lambda: [_HT_SPEC, _ES2_SPEC, _ED_SPEC, _GMAX_SPEC]


def _tc_first(x, w_aug, fout):
  f = functools.partial(_tc_first_body, fout=fout)
  return pl.pallas_call(
      f, grid=(GRID,),
      in_specs=[_X_SPEC(x.shape[1]), _full(w_aug.shape)],
      out_specs=_HAUG_SPECS(),
      out_shape=_HAUG_OUT(),
  )(x, w_aug)


def _tc_mid(acc, den, b, w_aug, fin, fout, want_x=False):
  f = functools.partial(_tc_mid_body, fin=fin, fout=fout, want_x=want_x)
  out_specs = _HAUG_SPECS()
  out_shape = _HAUG_OUT()
  if want_x:
    out_specs.append(_X_SPEC(fin))
    out_shape.append(jax.ShapeDtypeStruct((NN, fin), F32))
  return pl.pallas_call(
      f, grid=(GRID,),
      in_specs=[_ACC_SPEC, _DEN_SPEC, _full(b.shape), _full(w_aug.shape)],
      out_specs=out_specs,
      out_shape=out_shape,
  )(acc, den, b, w_aug)


def _tc_mask3(acc, den, b, latent, w_aug, fout):
  f = functools.partial(_tc_mask3_body, fout=fout)
  out_specs = _HAUG_SPECS() + [_X_SPEC(64), _X_SPEC(64)]
  out_shape = _HAUG_OUT() + [jax.ShapeDtypeStruct((NN, 64), F32),
                             jax.ShapeDtypeStruct((NN, 64), F32)]
  return pl.pallas_call(
      f, grid=(GRID,),
      in_specs=[_ACC_SPEC, _DEN_SPEC, _full(b.shape), _X_SPEC(64),
                _full(w_aug.shape)],
      out_specs=out_specs,
      out_shape=out_shape,
  )(acc, den, b, latent, w_aug)


def _tc_last(acc, den, b, fin):
  f = functools.partial(_tc_last_body, fin=fin)
  return pl.pallas_call(
      f, grid=(GRID,),
      in_specs=[_ACC_SPEC, _DEN_SPEC, _full(b.shape)],
      out_specs=_X_SPEC(fin),
      out_shape=jax.ShapeDtypeStruct((NN, fin), F32),
  )(acc, den, b)


def _tc_pool(msk, batch):
  return pl.pallas_call(
      _tc_pool_body, grid=(GRID,),
      in_specs=[_X_SPEC(64), pl.BlockSpec((1, 1, RB), lambda i: (i, 0, 0))],
      out_specs=pl.BlockSpec((NG, 65), lambda i: (0, 0)),
      out_shape=jax.ShapeDtypeStruct((NG, 65), F32),
  )(msk, batch)


def _tc_head(pool, w1, b1, w2, b2):
  return pl.pallas_call(
      _tc_head_body, grid=(1,),
      in_specs=[_full(pool.shape), _full(w1.shape), _full(b1.shape),
                _full(w2.shape), _full(b2.shape)],
      out_specs=_full((NG, 11)),
      out_shape=jax.ShapeDtypeStruct((NG, 11), F32),
  )(pool, w1, b1, w2, b2)


def _sc_edge_body(ht, es2, ed, gmax, ei, z32, z1,
                  acc_out, den_out,
                  srcva, dstva, esga, edga, rowga, pva,
                  srcvb, dstvb, esgb, edgb, rowgb, pvb, gmx,
                  acc_sh, den_sh, gsem, ssem):
  c = lax.axis_index("c")
  s = lax.axis_index("s")
  row0 = s * RPS

  @pl.when(s < 15)
  def _():
    pltpu.sync_copy(z32, acc_sh.at[pl.ds(row0, RPS), :])
    pltpu.sync_copy(z1, den_sh.at[pl.ds(row0, RPS), :])

  @pl.when(s == 15)
  def _():
    pltpu.sync_copy(z32.at[pl.ds(0, RPS_LAST), :],
                    acc_sh.at[pl.ds(row0, RPS_LAST), :])
    pltpu.sync_copy(z1.at[pl.ds(0, RPS_LAST), :],
                    den_sh.at[pl.ds(row0, RPS_LAST), :])

  pltpu.sync_copy(gmax, gmx)
  plsc.subcore_barrier()

  gm = gmx[...]
  coff = c * NN
  ntrip = jnp.where(s < (NCHUNK % NSUB), NCHUNK // NSUB + 1, NCHUNK // NSUB)
  bufs = [(srcva, dstva, esga, edga, rowga, pva),
          (srcvb, dstvb, esgb, edgb, rowgb, pvb)][:NSC]

  def chunk_body(k, carry):
    off = (s + NSUB * k) * SK
    for j, (sv, dv, _, _, _, _) in enumerate(bufs):
      pltpu.sync_copy(ei.at[0, pl.ds(off + j * CK, CK)], sv)
      pltpu.sync_copy(ei.at[1, pl.ds(off + j * CK, CK)], dv)
    for sv, _, _, _, _, _ in bufs:
      for g in range(CK // 16):
        sl = pl.ds(g * 16, 16)
        sv[sl] = sv[sl] + coff
    ds_ = []
    for sv, dv, esg, edg, rowg, _ in bufs:
      ds_.append(pltpu.async_copy(es2.at[sv], esg, gsem))
      ds_.append(pltpu.async_copy(ed.at[dv], edg, gsem))
      ds_.append(pltpu.async_copy(ht.at[sv], rowg, gsem))
    for d in ds_:
      d.wait()
    zcol = jnp.zeros((16,), I32)
    base = lax.iota(I32, 16)
    for _, _, esg, edg, rowg, pv in bufs:
      for g in range(CK // 16):
        sl = pl.ds(g * 16, 16)
        a = esg[sl]
        bb = edg[sl]
        t = a + bb
        e = jnp.maximum(t, NEG * t)
        u = gm + bb
        cb = jnp.maximum(u, NEG * u)
        p = jnp.exp(e - cb)
        plsc.store_scatter(pv, [base + (g * 16), zcol], p)

      def row_body(r, carry2, rowg=rowg, pv=pv):
        bc = plsc.load_gather(pv, [jnp.full((16,), r, I32),
                                   jnp.zeros((16,), I32)])
        rowg[r, pl.ds(0, 16)] = rowg[r, pl.ds(0, 16)] * bc
        rowg[r, pl.ds(16, 16)] = rowg[r, pl.ds(16, 16)] * bc
        return carry2

      lax.fori_loop(0, CK, row_body, 0)
    ds_ = []
    for _, dv, _, _, rowg, pv in bufs:
      ds_.append(pltpu.async_copy(pv, den_sh.at[dv], ssem, add=True))
      ds_.append(pltpu.async_copy(rowg, acc_sh.at[dv], ssem, add=True))
    for d in ds_:
      d.wait()
    return carry

  lax.fori_loop(0, ntrip, chunk_body, 0)
  plsc.subcore_barrier()
  out0 = coff + row0

  @pl.when(s < 15)
  def _():
    pltpu.sync_copy(acc_sh.at[pl.ds(row0, RPS), :],
                    acc_out.at[pl.ds(out0, RPS), :])
    pltpu.sync_copy(den_sh.at[pl.ds(row0, RPS), :],
                    den_out.at[pl.ds(out0, RPS), :])

  @pl.when(s == 15)
  def _():
    pltpu.sync_copy(acc_sh.at[pl.ds(row0, RPS_LAST), :],
                    acc_out.at[pl.ds(out0, RPS_LAST), :])
    pltpu.sync_copy(den_sh.at[pl.ds(row0, RPS_LAST), :],
                    den_out.at[pl.ds(out0, RPS_LAST), :])


@functools.partial(jax.jit, static_argnums=())
def _sc_edge_call(ht, es2, ed, gmax, ei, z32, z8):
  mesh = plsc.VectorSubcoreMesh(core_axis_name="c", subcore_axis_name="s")
  kern = pl.kernel(
      _sc_edge_body,
      out_type=[jax.ShapeDtypeStruct((2 * NN, 32), F32),
                jax.ShapeDtypeStruct((2 * NN, 1), F32)],
      mesh=mesh,
      compiler_params=pltpu.CompilerParams(needs_layout_passes=False,
                                           use_tc_tiling_on_sc=False),
      scratch_types=[
          pltpu.VMEM((CK,), I32),
          pltpu.VMEM((CK,), I32),
          pltpu.VMEM((CK,), F32),
          pltpu.VMEM((CK,), F32),
          pltpu.VMEM((CK, 32), F32),
          pltpu.VMEM((CK, 1), F32),
          pltpu.VMEM((CK,), I32),
          pltpu.VMEM((CK,), I32),
          pltpu.VMEM((CK,), F32),
          pltpu.VMEM((CK,), F32),
          pltpu.VMEM((CK, 32), F32),
          pltpu.VMEM((CK, 1), F32),
          pltpu.VMEM((16,), F32),
          pltpu.VMEM_SHARED((NN, 32), F32),
          pltpu.VMEM_SHARED((NN, 1), F32),
          pltpu.SemaphoreType.DMA,
          pltpu.SemaphoreType.DMA,
      ],
  )
  return kern(ht, es2, ed, gmax, ei, z32, z8)


def _sc_edge(ht, es2, ed, gmax, ei, z32, z1):
  acc, den = _sc_edge_call(ht.reshape(2 * NN, 32), es2.reshape(2 * NN),
                           ed.reshape(NN), gmax, ei, z32, z1)
  return acc.reshape(2, NN, 32), den.reshape(2, NN)[0].reshape(GRID, 1, RB)


def _aug(p):
  w, a_src, a_dst, b = p
  return jnp.concatenate(
      [w, (w @ a_src)[:, None], (w @ a_dst)[:, None]], axis=1), b


def kernel(x, edge_index, y, batch, epoch, params):
  del y, epoch
  ei = edge_index.astype(I32)
  z32 = jnp.zeros((RPS, 32), F32)
  z1 = jnp.zeros((RPS, 1), F32)

  wa1, b1 = _aug(params['conv1'])
  wa2, b2 = _aug(params['conv2'])
  wa3, b3 = _aug(params['conv3'])
  wm1, bm1 = _aug(params['mask1'])
  wm2, bm2 = _aug(params['mask2'])
  wm3, bm3 = _aug(params['mask3'])
  wa4, b4 = _aug(params['conv4'])
  wa5, b5 = _aug(params['conv5'])
  wa6, b6 = _aug(params['conv6'])

  def edge(haug):
    ht, es2, ed, gmax = haug[0], haug[1], haug[2], haug[3]
    return _sc_edge(ht, es2, ed, gmax, ei, z32, z1)

  o = _tc_first(x, wa1, 64)
  acc, den = edge(o)
  o = _tc_mid(acc, den, b1, wa2, 64, 64)
  acc, den = edge(o)
  o = _tc_mid(acc, den, b2, wa3, 64, 64)
  acc, den = edge(o)
  o = _tc_mid(acc, den, b3, wm1, 64, 32, want_x=True)
  latent = o[4]
  acc, den = edge(o)
  o = _tc_mid(acc, den, bm1, wm2, 32, 32)
  acc, den = edge(o)
  o = _tc_mid(acc, den, bm2, wm3, 32, 2)
  acc, den = edge(o)
  o = _tc_mask3(acc, den, bm3, latent, wa4, 64)
  msk, out2 = o[4], o[5]
  acc, den = edge(o)
  o = _tc_mid(acc, den, b4, wa5, 64, 64)
  acc, den = edge(o)
  o = _tc_mid(acc, den, b5, wa6, 64, 5)
  acc, den = edge(o)
  d3 = _tc_last(acc, den, b6, 5)

  pool = _tc_pool(msk, batch.astype(I32).reshape(GRID, 1, RB))
  w1, bb1 = params['lin1']
  w2, bb2 = params['lin2']
  c = _tc_head(pool, w1, bb1, w2, bb2)
  return (d3, out2, c)
